# Initial kernel scaffold; baseline (speedup 1.0000x reference)
import jax
import jax.numpy as jnp
from jax.experimental import pallas as pl


def kernel(li_bev_feats, li_bev_coors, ra_bev_feats, ra_bev_coors, pos_embedding, in_proj_w1, in_proj_b1, out_proj_w1, out_proj_b1, in_proj_w2, in_proj_b2, out_proj_w2, out_proj_b2):
    raise NotImplementedError("write your pallas kernel here")


# speedup vs baseline: 5.9985x; 5.9985x over previous
import numpy as np
import jax
import jax.numpy as jnp
from jax.experimental import pallas as pl

_SHIFTS = [(0, 0), (-1, 0), (1, 0), (0, 1), (-1, 1), (1, 1), (0, -1), (-1, -1), (1, -1)]
_H = 256
_W = 256
_E = 32
_HD = 16
_HP = _H + 2
_WP = _W + 2
_CELLS = _HP * _WP
_BN = 1024
_CT = 66 * _BN
_GRID = _CT // _BN


def _fuse_kernel(q_ref, qocc_ref, kp_ref, kc_ref, kn_ref, op_ref, oc_ref, on_ref,
                 pos_ref, wq_ref, wk_ref, wv_ref, bq_ref, bk_ref, bv_ref,
                 wo_ref, bo_ref, out_ref):
    kbuf = jnp.concatenate([kp_ref[...], kc_ref[...], kn_ref[...]], axis=0)
    obuf = jnp.concatenate([op_ref[...], oc_ref[...], on_ref[...]], axis=0)
    q2 = jnp.dot(q_ref[...], wq_ref[...], preferred_element_type=jnp.float32) + bq_ref[...]
    l_cols = []
    for dy, dx in _SHIFTS:
        lo = _BN + dy * _WP + dx
        ks = kbuf[lo:lo + _BN, :]
        k2 = jnp.dot(ks, wk_ref[...], preferred_element_type=jnp.float32) + bk_ref[...]
        prod = q2 * k2
        l0 = jnp.sum(prod[:, :_HD], axis=1, keepdims=True)
        l1 = jnp.sum(prod[:, _HD:], axis=1, keepdims=True)
        l_cols.append((l0, l1))
    scale = 1.0 / np.sqrt(_HD)
    attns = []
    for h in range(2):
        lh = jnp.concatenate([l_cols[s][h] for s in range(9)], axis=1) * scale
        m = jnp.max(lh, axis=1, keepdims=True)
        e = jnp.exp(lh - m)
        attns.append(e / jnp.sum(e, axis=1, keepdims=True))
    acc0 = jnp.zeros((_BN, _HD), jnp.float32)
    acc1 = jnp.zeros((_BN, _HD), jnp.float32)
    for s, (dy, dx) in enumerate(_SHIFTS):
        lo = _BN + dy * _WP + dx
        ks = kbuf[lo:lo + _BN, :]
        occ = obuf[lo:lo + _BN, :]
        posv = jnp.dot(pos_ref[s:s + 1, :], wv_ref[...], preferred_element_type=jnp.float32)
        v2 = jnp.dot(ks, wv_ref[...], preferred_element_type=jnp.float32) + bv_ref[...]
        v2 = jnp.where(occ > 0.0, v2 + posv, v2)
        acc0 = acc0 + attns[0][:, s:s + 1] * v2[:, :_HD]
        acc1 = acc1 + attns[1][:, s:s + 1] * v2[:, _HD:]
    o = jnp.concatenate([acc0, acc1], axis=1)
    o = jnp.dot(o, wo_ref[...], preferred_element_type=jnp.float32) + bo_ref[...]
    out_ref[...] = jnp.where(qocc_ref[...] > 0.0, o, 0.0)


def _run_fuse(qcanvas, qocc, kg, og, pos, wqT, wkT, wvT, bq, bk, bv, woT, bo):
    whole = lambda arr: pl.BlockSpec(arr.shape, lambda i: tuple(0 for _ in arr.shape))
    kblk = lambda j: pl.BlockSpec((_BN, _E), lambda i, j=j: (i + j, 0))
    oblk = lambda j: pl.BlockSpec((_BN, 1), lambda i, j=j: (i + j, 0))
    return pl.pallas_call(
        _fuse_kernel,
        grid=(_GRID,),
        in_specs=[
            pl.BlockSpec((_BN, _E), lambda i: (i, 0)),
            pl.BlockSpec((_BN, 1), lambda i: (i, 0)),
            kblk(0), kblk(1), kblk(2),
            oblk(0), oblk(1), oblk(2),
            whole(pos),
            whole(wqT), whole(wkT), whole(wvT),
            whole(bq), whole(bk), whole(bv),
            whole(woT), whole(bo),
        ],
        out_specs=pl.BlockSpec((_BN, _E), lambda i: (i, 0)),
        out_shape=jax.ShapeDtypeStruct((_CT, _E), jnp.float32),
    )(qcanvas, qocc, kg, kg, kg, og, og, og, pos, wqT, wkT, wvT, bq, bk, bv, woT, bo)


def _build(feats, coors):
    c = coors.astype(jnp.int32)
    flat = (c[:, 0] + 1) * _WP + (c[:, 1] + 1)
    canvas = jnp.zeros((_CT, _E), jnp.float32).at[flat].set(feats)
    occ = jnp.zeros((_CT,), jnp.float32).at[flat].set(1.0)[:, None]
    kg = jnp.pad(canvas, ((_BN, _BN), (0, 0)))
    og = jnp.pad(occ, ((_BN, _BN), (0, 0)))
    return canvas, occ, kg, og


def _extract(o):
    img = o[:_CELLS].reshape(_HP, _WP, _E)[1:_H + 1, 1:_W + 1]
    return jnp.transpose(img, (2, 0, 1))[None]


def kernel(li_bev_feats, li_bev_coors, ra_bev_feats, ra_bev_coors, pos_embedding,
           in_proj_w1, in_proj_b1, out_proj_w1, out_proj_b1,
           in_proj_w2, in_proj_b2, out_proj_w2, out_proj_b2):
    def wpack(wi, bi, wo, bo):
        return (wi[:_E].T, wi[_E:2 * _E].T, wi[2 * _E:].T,
                bi[:_E][None], bi[_E:2 * _E][None], bi[2 * _E:][None],
                wo.T, bo[None])

    w1 = wpack(in_proj_w1, in_proj_b1, out_proj_w1, out_proj_b1)
    w2 = wpack(in_proj_w2, in_proj_b2, out_proj_w2, out_proj_b2)
    li_out, ra_out = [], []
    for b in range(li_bev_feats.shape[0]):
        lq, locc, lkg, log_ = _build(li_bev_feats[b], li_bev_coors[b])
        rq, rocc, rkg, rog = _build(ra_bev_feats[b], ra_bev_coors[b])
        o1 = _run_fuse(lq, locc, rkg, rog, pos_embedding, *w1)
        o2 = _run_fuse(rq, rocc, lkg, log_, pos_embedding, *w2)
        li_out.append(_extract(o1))
        ra_out.append(_extract(o2))
    return jnp.concatenate(li_out, axis=0), jnp.concatenate(ra_out, axis=0)
